# baseline (speedup 1.0000x reference)
import jax
import jax.numpy as jnp
from jax import lax
from jax.experimental import pallas as pl


N_OBJ = 5000
NUM_OBJ_CLS = 151
N_REL = 20000
REL_DIM = 4096
NUM_REL_CLS = 51

GRID_M = 20
GRID_K = 4
BM = N_REL // GRID_M
BK = REL_DIM // GRID_K


def _mm_body(vr_ref, w_ref, b_ref, out_ref):
    k = pl.program_id(1)
    acc = lax.dot_general(
        vr_ref[...], w_ref[...],
        (((1,), (1,)), ((), ())),
        preferred_element_type=jnp.float32,
    )

    @pl.when(k == 0)
    def _():
        out_ref[...] = acc + b_ref[...]

    @pl.when(k != 0)
    def _():
        out_ref[...] += acc


@jax.jit
def kernel(obj_logits, vr, W, b):
    b2 = b.reshape(1, NUM_REL_CLS)
    rel_dists = pl.pallas_call(
        _mm_body,
        grid=(GRID_M, GRID_K),
        in_specs=[
            pl.BlockSpec((BM, BK), lambda i, k: (i, k)),
            pl.BlockSpec((NUM_REL_CLS, BK), lambda i, k: (0, k)),
            pl.BlockSpec((1, NUM_REL_CLS), lambda i, k: (0, 0)),
        ],
        out_specs=pl.BlockSpec((BM, NUM_REL_CLS), lambda i, k: (i, 0)),
        out_shape=jax.ShapeDtypeStruct((N_REL, NUM_REL_CLS), jnp.float32),
    )(vr, W, b2)
    obj_preds = jnp.zeros((N_OBJ,), jnp.int32)
    return obj_logits, obj_preds, rel_dists


# speedup vs baseline: 1.0032x; 1.0032x over previous
import functools

import jax
import jax.numpy as jnp
from jax import lax
from jax.experimental import pallas as pl
from jax.experimental.pallas import tpu as pltpu
from jax.experimental.pallas import tpu_sc as plsc


N_OBJ = 5000
NUM_OBJ_CLS = 151
N_REL = 20000
REL_DIM = 4096
NUM_REL_CLS = 51

GRID = 20
BM = N_REL // GRID

NW = 32
RPW = 160
LAST_BASE = N_OBJ - RPW


def _mm_body(vr_ref, w_ref, b_ref, out_ref):
    acc = lax.dot_general(
        vr_ref[...], w_ref[...],
        (((1,), (1,)), ((), ())),
        preferred_element_type=jnp.float32,
    )
    out_ref[...] = acc + b_ref[...]


def _tc_matmul(vr, W, b2):
    return pl.pallas_call(
        _mm_body,
        grid=(GRID,),
        in_specs=[
            pl.BlockSpec((BM, REL_DIM), lambda i: (i, 0)),
            pl.BlockSpec((NUM_REL_CLS, REL_DIM), lambda i: (0, 0)),
            pl.BlockSpec((1, NUM_REL_CLS), lambda i: (0, 0)),
        ],
        out_specs=pl.BlockSpec((BM, NUM_REL_CLS), lambda i: (i, 0)),
        out_shape=jax.ShapeDtypeStruct((N_REL, NUM_REL_CLS), jnp.float32),
    )(vr, W, b2)


def _sc_argmax_body(obj_hbm, out_hbm, block_v, preds_v):
    wid = lax.axis_index("s") * 2 + lax.axis_index("c")
    base = jnp.minimum(wid * RPW, LAST_BASE)
    pltpu.sync_copy(obj_hbm.at[pl.ds(base * NUM_OBJ_CLS, RPW * NUM_OBJ_CLS)], block_v)
    lanes = lax.iota(jnp.int32, 16)
    for g in range(RPW // 16):
        rows = lanes + (g * 16)

        def step(c, carry):
            m, mi, col = carry
            v = plsc.load_gather(block_v, [rows * NUM_OBJ_CLS + col])
            upd = v > m
            return (
                jnp.where(upd, v, m),
                jnp.where(upd, col, mi),
                col + jnp.ones((16,), jnp.int32),
            )

        m0 = jnp.full((16,), -jnp.inf, jnp.float32)
        i0 = jnp.zeros((16,), jnp.int32)
        c0 = jnp.ones((16,), jnp.int32)
        _, mi, _ = lax.fori_loop(0, NUM_OBJ_CLS - 1, step, (m0, i0, c0))
        preds_v[pl.ds(g * 16, 16)] = mi
    pltpu.sync_copy(preds_v, out_hbm.at[pl.ds(base, RPW)])


@functools.partial(
    pl.kernel,
    out_type=jax.ShapeDtypeStruct((N_OBJ,), jnp.int32),
    mesh=plsc.VectorSubcoreMesh(core_axis_name="c", subcore_axis_name="s"),
    scratch_types=[
        pltpu.VMEM((RPW * NUM_OBJ_CLS,), jnp.float32),
        pltpu.VMEM((RPW,), jnp.int32),
    ],
    compiler_params=pltpu.CompilerParams(needs_layout_passes=False),
)
def _sc_argmax(obj_hbm, out_hbm, block_v, preds_v):
    _sc_argmax_body(obj_hbm, out_hbm, block_v, preds_v)


@jax.jit
def kernel(obj_logits, vr, W, b):
    b2 = b.reshape(1, NUM_REL_CLS)
    obj_preds = _sc_argmax(obj_logits.reshape(N_OBJ * NUM_OBJ_CLS))
    rel_dists = _tc_matmul(vr, W, b2)
    return obj_logits, obj_preds, rel_dists


# speedup vs baseline: 1.0399x; 1.0366x over previous
import functools

import jax
import jax.numpy as jnp
from jax import lax
from jax.experimental import pallas as pl
from jax.experimental.pallas import tpu as pltpu
from jax.experimental.pallas import tpu_sc as plsc


N_OBJ = 5000
NUM_OBJ_CLS = 151
N_REL = 20000
REL_DIM = 4096
NUM_REL_CLS = 51

GRID = 20
BM = N_REL // GRID

NW = 32
RPW = 160
LAST_BASE = N_OBJ - RPW


def _mm_body(vr_ref, w_ref, b_ref, out_ref):
    acc = lax.dot_general(
        vr_ref[...], w_ref[...],
        (((1,), (1,)), ((), ())),
        preferred_element_type=jnp.float32,
    )
    out_ref[...] = acc + b_ref[...]


def _tc_matmul(vr, W, b2):
    return pl.pallas_call(
        _mm_body,
        grid=(GRID,),
        in_specs=[
            pl.BlockSpec((BM, REL_DIM), lambda i: (i, 0)),
            pl.BlockSpec((NUM_REL_CLS, REL_DIM), lambda i: (0, 0)),
            pl.BlockSpec((1, NUM_REL_CLS), lambda i: (0, 0)),
        ],
        out_specs=pl.BlockSpec((BM, NUM_REL_CLS), lambda i: (i, 0)),
        out_shape=jax.ShapeDtypeStruct((N_REL, NUM_REL_CLS), jnp.float32),
    )(vr, W, b2)


def _sc_argmax_body(obj_hbm, out_hbm, block_v, preds_v):
    wid = lax.axis_index("s") * 2 + lax.axis_index("c")
    base = jnp.minimum(wid * RPW, LAST_BASE)
    pltpu.sync_copy(obj_hbm.at[pl.ds(base, RPW)], block_v)
    lanes = lax.iota(jnp.int32, 16)
    for g in range(RPW // 16):
        rows = lanes + (g * 16)

        def step(c, carry):
            m, mi, col = carry
            v = plsc.load_gather(block_v, [rows, col])
            upd = v > m
            return (
                jnp.where(upd, v, m),
                jnp.where(upd, col, mi),
                col + jnp.ones((16,), jnp.int32),
            )

        m0 = jnp.full((16,), -jnp.inf, jnp.float32)
        i0 = jnp.zeros((16,), jnp.int32)
        c0 = jnp.ones((16,), jnp.int32)
        _, mi, _ = lax.fori_loop(0, NUM_OBJ_CLS - 1, step, (m0, i0, c0))
        preds_v[pl.ds(g * 16, 16)] = mi
    pltpu.sync_copy(preds_v, out_hbm.at[pl.ds(base, RPW)])


@functools.partial(
    pl.kernel,
    out_type=jax.ShapeDtypeStruct((N_OBJ,), jnp.int32),
    mesh=plsc.VectorSubcoreMesh(core_axis_name="c", subcore_axis_name="s"),
    scratch_types=[
        pltpu.VMEM((RPW, NUM_OBJ_CLS), jnp.float32),
        pltpu.VMEM((RPW,), jnp.int32),
    ],
    compiler_params=pltpu.CompilerParams(needs_layout_passes=False),
)
def _sc_argmax(obj_hbm, out_hbm, block_v, preds_v):
    _sc_argmax_body(obj_hbm, out_hbm, block_v, preds_v)


@jax.jit
def kernel(obj_logits, vr, W, b):
    b2 = b.reshape(1, NUM_REL_CLS)
    obj_preds = _sc_argmax(obj_logits)
    rel_dists = _tc_matmul(vr, W, b2)
    return obj_logits, obj_preds, rel_dists


# speedup vs baseline: 1.1696x; 1.1248x over previous
import functools

import jax
import jax.numpy as jnp
from jax import lax
from jax.experimental import pallas as pl
from jax.experimental.pallas import tpu as pltpu
from jax.experimental.pallas import tpu_sc as plsc


N_OBJ = 5000
NUM_OBJ_CLS = 151
N_REL = 20000
REL_DIM = 4096
NUM_REL_CLS = 51

BM = 1024
GRID = (N_REL + BM - 1) // BM

SLICE = 128
NSLICES = 40
N_OBJ_PAD = NSLICES * SLICE


def _mm_body(vr_ref, w_ref, b_ref, out_ref):
    acc = lax.dot_general(
        w_ref[...], vr_ref[...],
        (((1,), (1,)), ((), ())),
        preferred_element_type=jnp.float32,
    )
    out_ref[...] = acc + b_ref[...]


def _tc_matmul_t(vr, W, b_col):
    return pl.pallas_call(
        _mm_body,
        grid=(GRID,),
        in_specs=[
            pl.BlockSpec((BM, REL_DIM), lambda i: (i, 0)),
            pl.BlockSpec((NUM_REL_CLS, REL_DIM), lambda i: (0, 0)),
            pl.BlockSpec((NUM_REL_CLS, 1), lambda i: (0, 0)),
        ],
        out_specs=pl.BlockSpec((NUM_REL_CLS, BM), lambda i: (0, i)),
        out_shape=jax.ShapeDtypeStruct((NUM_REL_CLS, N_REL), jnp.float32),
    )(vr, W, b_col)


def _sc_slice(objt_hbm, preds_hbm, block_v, preds_v, s):
    base = s * SLICE
    pltpu.sync_copy(objt_hbm.at[:, pl.ds(base, SLICE)], block_v)
    for g in range(SLICE // 16):
        def step(c, carry):
            m, mi, col = carry
            v = block_v[c, pl.ds(g * 16, 16)]
            upd = v > m
            return (
                jnp.where(upd, v, m),
                jnp.where(upd, col, mi),
                col + jnp.ones((16,), jnp.int32),
            )

        m0 = jnp.full((16,), -jnp.inf, jnp.float32)
        i0 = jnp.zeros((16,), jnp.int32)
        c0 = jnp.ones((16,), jnp.int32)
        _, mi, _ = lax.fori_loop(1, NUM_OBJ_CLS, step, (m0, i0, c0))
        preds_v[pl.ds(g * 16, 16)] = mi
    pltpu.sync_copy(preds_v, preds_hbm.at[pl.ds(base, SLICE)])


def _sc_argmax_body(objt_hbm, preds_hbm, block_v, preds_v):
    wid = lax.axis_index("s") * 2 + lax.axis_index("c")
    _sc_slice(objt_hbm, preds_hbm, block_v, preds_v, wid)

    @pl.when(wid < NSLICES - 32)
    def _():
        _sc_slice(objt_hbm, preds_hbm, block_v, preds_v, wid + 32)


@functools.partial(
    pl.kernel,
    out_type=jax.ShapeDtypeStruct((N_OBJ_PAD,), jnp.int32),
    mesh=plsc.VectorSubcoreMesh(core_axis_name="c", subcore_axis_name="s"),
    scratch_types=[
        pltpu.VMEM((NUM_OBJ_CLS, SLICE), jnp.float32),
        pltpu.VMEM((SLICE,), jnp.int32),
    ],
    compiler_params=pltpu.CompilerParams(needs_layout_passes=False),
)
def _sc_argmax(objt_hbm, preds_hbm, block_v, preds_v):
    _sc_argmax_body(objt_hbm, preds_hbm, block_v, preds_v)


@jax.jit
def kernel(obj_logits, vr, W, b):
    b_col = b.reshape(NUM_REL_CLS, 1)
    obj_preds = _sc_argmax(obj_logits.T)[:N_OBJ]
    rel_t = _tc_matmul_t(vr, W, b_col)
    rel_dists = rel_t.T
    return obj_logits, obj_preds, rel_dists
